# baseline (speedup 1.0000x reference)
import jax
import jax.numpy as jnp
from jax import lax
from jax.experimental import pallas as pl
from jax.experimental.pallas import tpu as pltpu


def kernel(X1, X2, A, a, gamma, graph):
    n1, Lx = X1.shape
    n2 = X2.shape[0]
    ns, d = A.shape
    nh = Lx // 2

    def body(x1_ref, x2_ref, a_ref, a2_ref, asm_ref, as_ref, gs_ref, out_ref,
             oh1o_s, oh2o_s, oh1p_s, oh2p_s, r1so_s, r1sp_s,
             x1t_s, x2t_s, acc0_s, acc1_s):
        A_v = a_ref[:]
        A2_v = a2_ref[:]
        PS = a2_ref.shape[0] // 2
        S = lax.dot_general(A_v, A_v, (((1,), (1,)), ((), ())),
                            preferred_element_type=jnp.float32)
        S_bf = S.astype(jnp.bfloat16)
        S2 = lax.dot_general(A2_v, A2_v, (((1,), (1,)), ((), ())),
                             preferred_element_type=jnp.float32)
        ri = lax.broadcasted_iota(jnp.int32, (2 * PS, 2 * PS), 0)
        ci = lax.broadcasted_iota(jnp.int32, (2 * PS, 2 * PS), 1)
        S2bd = jnp.where((ri < PS) == (ci < PS), S2, 0.0).astype(jnp.bfloat16)

        x1t_s[:] = jnp.transpose(x1_ref[:].astype(jnp.float32))
        x2t_s[:] = jnp.transpose(x2_ref[:].astype(jnp.float32))

        iota1 = lax.broadcasted_iota(jnp.int32, (ns, n1), 0).astype(jnp.float32)
        iota2 = lax.broadcasted_iota(jnp.int32, (ns, n2), 0).astype(jnp.float32)
        iota1p = lax.broadcasted_iota(jnp.int32, (PS, n1), 0).astype(jnp.float32)
        iota2p = lax.broadcasted_iota(jnp.int32, (PS, n2), 0).astype(jnp.float32)

        def oh1_at(i):
            return (iota1 == x1t_s[i:i + 1, :]).astype(jnp.bfloat16)

        def oh2_at(i):
            return (iota2 == x2t_s[i:i + 1, :]).astype(jnp.bfloat16)

        def oh1p_at(i):
            return (iota1p == x1t_s[i:i + 1, :]).astype(jnp.bfloat16)

        def oh2p_at(i):
            return (iota2p == x2t_s[i:i + 1, :]).astype(jnp.bfloat16)

        for jj in range(nh):
            j = 2 * jj + 1
            c1 = slice(jj * n1, (jj + 1) * n1)
            c2 = slice(jj * n2, (jj + 1) * n2)
            oh1o_s[:, c1] = oh1_at(j)
            oh2o_s[:, c2] = oh2_at(j)
            oh1p_s[0:PS, c1] = oh1p_at(j - 1)
            oh2p_s[0:PS, c2] = oh2p_at(j - 1)
            if j + 1 < Lx:
                oh1p_s[PS:2 * PS, c1] = oh1p_at(j + 1)
                oh2p_s[PS:2 * PS, c2] = oh2p_at(j + 1)
            else:
                oh1p_s[PS:2 * PS, c1] = jnp.zeros((PS, n1), jnp.bfloat16)
                oh2p_s[PS:2 * PS, c2] = jnp.zeros((PS, n2), jnp.bfloat16)

        r1so_s[:] = lax.dot_general(S_bf, oh1o_s[:], (((0,), (0,)), ((), ())),
                                    preferred_element_type=jnp.float32
                                    ).astype(jnp.bfloat16)
        r1sp_s[:] = lax.dot_general(S2bd, oh1p_s[:], (((0,), (0,)), ((), ())),
                                    preferred_element_type=jnp.float32
                                    ).astype(jnp.bfloat16)

        acc0_s[:] = jnp.zeros((n1, n2), jnp.float32)
        acc1_s[:] = jnp.zeros((n1, n2), jnp.float32)
        for jj in range(nh):
            c1 = slice(jj * n1, (jj + 1) * n1)
            c2 = slice(jj * n2, (jj + 1) * n2)
            t = lax.dot_general(r1so_s[:, c1], oh2o_s[:, c2],
                                (((0,), (0,)), ((), ())),
                                preferred_element_type=jnp.float32)
            v = lax.dot_general(r1sp_s[:, c1], oh2p_s[:, c2],
                                (((0,), (0,)), ((), ())),
                                preferred_element_type=jnp.float32)
            if jj % 2:
                acc0_s[:] = acc0_s[:] + t * v
            else:
                acc1_s[:] = acc1_s[:] + t * v
        k0 = 2.0 * (acc0_s[:] + acc1_s[:])

        dcs = []
        for c in range(ns):
            s = asm_ref[c, 0] * asm_ref[c, 0]
            for k in range(1, d):
                s = s + asm_ref[c, k] * asm_ref[c, k]
            dcs.append(s)

        def dmat_of(xt_s, n):
            xt = xt_s[:]
            m = jnp.full((Lx, n), dcs[0], jnp.float32)
            for c in range(1, ns):
                m = jnp.where(xt == jnp.float32(c), dcs[c], m)
            return m

        dm1 = dmat_of(x1t_s, n1)
        dm2 = dmat_of(x2t_s, n2)
        k1row = jnp.sum(dm1[0:Lx - 1, :] * dm1[1:Lx, :], axis=0, keepdims=True)
        k2row = jnp.sum(dm2[0:Lx - 1, :] * dm2[1:Lx, :], axis=0, keepdims=True)
        k1c = jnp.transpose(2.0 * k1row)
        k2r = 2.0 * k2row
        ratio = k0 / jnp.sqrt(k1c) / jnp.sqrt(k2r)

        av = as_ref[0, 0]
        gv = gs_ref[0, 0]
        powed = jnp.where(gv == jnp.float32(1.0), ratio,
                          jnp.exp(gv * jnp.log(ratio)))
        out_ref[:] = (av * av) * powed

    PS = 32
    Af = A.astype(jnp.float32)
    A2 = jnp.concatenate(
        [Af, jnp.zeros((PS - ns, d), jnp.float32),
         Af, jnp.zeros((PS - ns, d), jnp.float32)], axis=0)
    return pl.pallas_call(
        body,
        out_shape=jax.ShapeDtypeStruct((n1, n2), jnp.float32),
        in_specs=[
            pl.BlockSpec(memory_space=pltpu.VMEM),
            pl.BlockSpec(memory_space=pltpu.VMEM),
            pl.BlockSpec(memory_space=pltpu.VMEM),
            pl.BlockSpec(memory_space=pltpu.VMEM),
            pl.BlockSpec(memory_space=pltpu.SMEM),
            pl.BlockSpec(memory_space=pltpu.SMEM),
            pl.BlockSpec(memory_space=pltpu.SMEM),
        ],
        scratch_shapes=[
            pltpu.VMEM((ns, nh * n1), jnp.bfloat16),
            pltpu.VMEM((ns, nh * n2), jnp.bfloat16),
            pltpu.VMEM((2 * PS, nh * n1), jnp.bfloat16),
            pltpu.VMEM((2 * PS, nh * n2), jnp.bfloat16),
            pltpu.VMEM((ns, nh * n1), jnp.bfloat16),
            pltpu.VMEM((2 * PS, nh * n1), jnp.bfloat16),
            pltpu.VMEM((Lx, n1), jnp.float32),
            pltpu.VMEM((Lx, n2), jnp.float32),
            pltpu.VMEM((n1, n2), jnp.float32),
            pltpu.VMEM((n1, n2), jnp.float32),
        ],
    )(X1, X2, Af, A2, Af, a.reshape(1, 1), gamma.reshape(1, 1))


# speedup vs baseline: 1.0243x; 1.0243x over previous
import jax
import jax.numpy as jnp
from jax import lax
from jax.experimental import pallas as pl
from jax.experimental.pallas import tpu as pltpu


def kernel(X1, X2, A, a, gamma, graph):
    n1, Lx = X1.shape
    n2 = X2.shape[0]
    ns, d = A.shape
    nh = Lx // 2
    PS = 32

    def body(x1_ref, x2_ref, a_ref, asm_ref, as_ref, gs_ref, out_ref,
             oh1_s, oh2o_s, oh2p_s, r1s_s, r1sp_s,
             x1t_s, x2t_s, acc0_s, acc1_s):
        A_v = a_ref[:]
        S = lax.dot_general(A_v, A_v, (((1,), (1,)), ((), ())),
                            preferred_element_type=jnp.float32)
        S_bf = S.astype(jnp.bfloat16)

        x1t_s[:] = jnp.transpose(x1_ref[:].astype(jnp.float32))
        x2t_s[:] = jnp.transpose(x2_ref[:].astype(jnp.float32))

        iota1 = lax.broadcasted_iota(jnp.int32, (ns, n1), 0).astype(jnp.float32)
        iota2 = lax.broadcasted_iota(jnp.int32, (ns, n2), 0).astype(jnp.float32)
        iota2p = lax.broadcasted_iota(jnp.int32, (PS, n2), 0).astype(jnp.float32)

        def oh1_at(i):
            return (iota1 == x1t_s[i:i + 1, :]).astype(jnp.bfloat16)

        def oh2_at(i):
            return (iota2 == x2t_s[i:i + 1, :]).astype(jnp.bfloat16)

        def oh2p_at(i):
            return (iota2p == x2t_s[i:i + 1, :]).astype(jnp.bfloat16)

        def blk(base, i, n):
            return slice(base + i * n, base + (i + 1) * n)

        for i in range(Lx):
            oh1_s[:, blk(0, i, n1)] = oh1_at(i)
            if i % 2:
                oh2o_s[:, blk(0, (i - 1) // 2, n2)] = oh2_at(i)
            else:
                o2 = oh2p_at(i)
                m = i // 2
                if m < nh:
                    oh2p_s[0:PS, blk(0, m, n2)] = o2
                if m >= 1:
                    oh2p_s[PS:2 * PS, blk(0, m - 1, n2)] = o2
        oh2p_s[PS:2 * PS, blk(0, nh - 1, n2)] = jnp.zeros((PS, n2), jnp.bfloat16)

        r1s_s[:] = lax.dot_general(S_bf, oh1_s[:], (((0,), (0,)), ((), ())),
                                   preferred_element_type=jnp.float32
                                   ).astype(jnp.bfloat16)

        zband = jnp.zeros((PS - ns, nh * n1), jnp.bfloat16)
        r1sp_s[ns:PS, :] = zband
        r1sp_s[PS + ns:2 * PS, :] = zband
        for m in range(nh):
            ev = r1s_s[:, blk(0, 2 * m, n1)]
            r1sp_s[0:ns, blk(0, m, n1)] = ev
            if m >= 1:
                r1sp_s[PS:PS + ns, blk(0, m - 1, n1)] = ev
        r1sp_s[PS:PS + ns, blk(0, nh - 1, n1)] = jnp.zeros((ns, n1), jnp.bfloat16)

        def t_at(jj):
            j = 2 * jj + 1
            return lax.dot_general(r1s_s[:, blk(0, j, n1)],
                                   oh2o_s[:, blk(0, jj, n2)],
                                   (((0,), (0,)), ((), ())),
                                   preferred_element_type=jnp.float32)

        def v_at(jj):
            return lax.dot_general(r1sp_s[:, blk(0, jj, n1)],
                                   oh2p_s[:, blk(0, jj, n2)],
                                   (((0,), (0,)), ((), ())),
                                   preferred_element_type=jnp.float32)

        acc0_s[:] = jnp.zeros((n1, n2), jnp.float32)
        acc1_s[:] = jnp.zeros((n1, n2), jnp.float32)
        for jj in range(0, nh, 2):
            acc0_s[:] = acc0_s[:] + t_at(jj) * v_at(jj)
            acc1_s[:] = acc1_s[:] + t_at(jj + 1) * v_at(jj + 1)
        k0 = 2.0 * (acc0_s[:] + acc1_s[:])

        dcs = []
        for c in range(ns):
            s = asm_ref[c, 0] * asm_ref[c, 0]
            for k in range(1, d):
                s = s + asm_ref[c, k] * asm_ref[c, k]
            dcs.append(s)

        def dmat_of(xt_s, n):
            xt = xt_s[:]
            m = jnp.full((Lx, n), dcs[0], jnp.float32)
            for c in range(1, ns):
                m = jnp.where(xt == jnp.float32(c), dcs[c], m)
            return m

        dm1 = dmat_of(x1t_s, n1)
        dm2 = dmat_of(x2t_s, n2)
        k1row = jnp.sum(dm1[0:Lx - 1, :] * dm1[1:Lx, :], axis=0, keepdims=True)
        k2row = jnp.sum(dm2[0:Lx - 1, :] * dm2[1:Lx, :], axis=0, keepdims=True)
        k1c = jnp.transpose(2.0 * k1row)
        k2r = 2.0 * k2row
        ratio = k0 / jnp.sqrt(k1c) / jnp.sqrt(k2r)

        av = as_ref[0, 0]
        gv = gs_ref[0, 0]
        powed = jnp.where(gv == jnp.float32(1.0), ratio,
                          jnp.exp(gv * jnp.log(ratio)))
        out_ref[:] = (av * av) * powed

    Af = A.astype(jnp.float32)
    return pl.pallas_call(
        body,
        out_shape=jax.ShapeDtypeStruct((n1, n2), jnp.float32),
        in_specs=[
            pl.BlockSpec(memory_space=pltpu.VMEM),
            pl.BlockSpec(memory_space=pltpu.VMEM),
            pl.BlockSpec(memory_space=pltpu.VMEM),
            pl.BlockSpec(memory_space=pltpu.SMEM),
            pl.BlockSpec(memory_space=pltpu.SMEM),
            pl.BlockSpec(memory_space=pltpu.SMEM),
        ],
        scratch_shapes=[
            pltpu.VMEM((ns, Lx * n1), jnp.bfloat16),
            pltpu.VMEM((ns, nh * n2), jnp.bfloat16),
            pltpu.VMEM((2 * PS, nh * n2), jnp.bfloat16),
            pltpu.VMEM((ns, Lx * n1), jnp.bfloat16),
            pltpu.VMEM((2 * PS, nh * n1), jnp.bfloat16),
            pltpu.VMEM((Lx, n1), jnp.float32),
            pltpu.VMEM((Lx, n2), jnp.float32),
            pltpu.VMEM((n1, n2), jnp.float32),
            pltpu.VMEM((n1, n2), jnp.float32),
        ],
    )(X1, X2, Af, Af, a.reshape(1, 1), gamma.reshape(1, 1))
